# Initial kernel scaffold; baseline (speedup 1.0000x reference)
import functools

import jax
import jax.numpy as jnp
from jax import lax
from jax.experimental import pallas as pl
from jax.experimental.pallas import tpu as pltpu
from jax.experimental.pallas import tpu_sc as plsc

_NSEG = 1024
_TGLOB = 0.5
_NC = 2
_NS = 16
_NW = _NC * _NS
_L = 16
_BLK = 8192


def _pad_to(x, n, val):
    if x.shape[0] == n:
        return x
    return jnp.concatenate([x, jnp.full((n - x.shape[0],), val, x.dtype)])


def _sc_partials(coeffs, bfi, cb, ai, ab, g_z, t_z, n_basis_p, n_atoms_p):
    b_chunk = n_basis_p // _NW
    a_chunk = n_atoms_p // _NW
    n_full, rem = divmod(b_chunk, _BLK)
    assert rem % _L == 0 and a_chunk % _L == 0 and a_chunk <= _BLK

    mesh = plsc.VectorSubcoreMesh(core_axis_name="c", subcore_axis_name="s")

    @functools.partial(
        pl.kernel,
        mesh=mesh,
        out_type=jax.ShapeDtypeStruct((_NW, _NSEG), jnp.float32),
        scratch_types=[
            pltpu.VMEM((64,), jnp.float32),
            pltpu.VMEM((16,), jnp.float32),
            pltpu.VMEM((_NSEG,), jnp.float32),
            pltpu.VMEM((_BLK,), jnp.float32),
            pltpu.VMEM((_BLK,), jnp.int32),
            pltpu.VMEM((_BLK,), jnp.int32),
            pltpu.VMEM((_BLK,), jnp.int32),
            pltpu.VMEM((_BLK,), jnp.int32),
        ],
    )
    def sc_kern(coeffs_hbm, bfi_hbm, cb_hbm, ai_hbm, ab_hbm, gz_hbm, tz_hbm,
                out_hbm, g_tab, t_tab, acc, c_v, b_v, s_v, ai_v, ab_v):
        wid = lax.axis_index("s") * _NC + lax.axis_index("c")

        pltpu.sync_copy(gz_hbm, g_tab)
        pltpu.sync_copy(tz_hbm, t_tab)

        def zero_step(i, carry):
            acc[pl.ds(i * _L, _L)] = jnp.zeros((_L,), jnp.float32)
            return carry
        lax.fori_loop(0, _NSEG // _L, zero_step, 0)

        def basis_vecs(j, carry):
            c = c_v[pl.ds(j * _L, _L)]
            b = b_v[pl.ds(j * _L, _L)]
            s = s_v[pl.ds(j * _L, _L)]
            g = plsc.load_gather(g_tab, [b])
            plsc.addupdate_scatter(acc, [s], g * c)
            return carry

        base = wid * b_chunk
        for blk in range(n_full):
            off = base + blk * _BLK
            pltpu.sync_copy(coeffs_hbm.at[pl.ds(off, _BLK)], c_v)
            pltpu.sync_copy(bfi_hbm.at[pl.ds(off, _BLK)], b_v)
            pltpu.sync_copy(cb_hbm.at[pl.ds(off, _BLK)], s_v)
            lax.fori_loop(0, _BLK // _L, basis_vecs, 0)
        if rem:
            off = base + n_full * _BLK
            pltpu.sync_copy(coeffs_hbm.at[pl.ds(off, rem)], c_v.at[pl.ds(0, rem)])
            pltpu.sync_copy(bfi_hbm.at[pl.ds(off, rem)], b_v.at[pl.ds(0, rem)])
            pltpu.sync_copy(cb_hbm.at[pl.ds(off, rem)], s_v.at[pl.ds(0, rem)])
            lax.fori_loop(0, rem // _L, basis_vecs, 0)

        a_base = wid * a_chunk
        pltpu.sync_copy(ai_hbm.at[pl.ds(a_base, a_chunk)], ai_v.at[pl.ds(0, a_chunk)])
        pltpu.sync_copy(ab_hbm.at[pl.ds(a_base, a_chunk)], ab_v.at[pl.ds(0, a_chunk)])

        def atom_vecs(j, carry):
            i = ai_v[pl.ds(j * _L, _L)]
            sgm = ab_v[pl.ds(j * _L, _L)]
            t = plsc.load_gather(t_tab, [i])
            plsc.addupdate_scatter(acc, [sgm], t)
            return carry
        lax.fori_loop(0, a_chunk // _L, atom_vecs, 0)

        pltpu.sync_copy(acc, out_hbm.at[wid])

    return sc_kern(coeffs, bfi, cb, ai, ab, g_z, t_z)


def _combine_body(p_ref, o_ref):
    o_ref[...] = jnp.sum(p_ref[...], axis=0, keepdims=True) + _TGLOB


def kernel(atom_ind, coeffs, basis_function_ind, atom_batch, coeffs_batch,
           t_z, g_z):
    n_basis = coeffs.shape[0]
    n_atoms = atom_ind.shape[0]
    grain = _NW * _L
    n_basis_p = -(-n_basis // grain) * grain
    n_atoms_p = -(-n_atoms // grain) * grain

    coeffs_p = _pad_to(coeffs, n_basis_p, 0.0)
    bfi_p = _pad_to(basis_function_ind, n_basis_p, 0)
    cb_p = _pad_to(coeffs_batch, n_basis_p, 0)
    ai_p = _pad_to(atom_ind, n_atoms_p, t_z.shape[0])
    ab_p = _pad_to(atom_batch, n_atoms_p, 0)
    gz_p = _pad_to(g_z, 64, 0.0)
    tz_p = _pad_to(t_z, 16, 0.0)

    partials = _sc_partials(coeffs_p, bfi_p, cb_p, ai_p, ab_p, gz_p, tz_p,
                            n_basis_p, n_atoms_p)

    out = pl.pallas_call(
        _combine_body,
        out_shape=jax.ShapeDtypeStruct((1, _NSEG), jnp.float32),
    )(partials)
    return out.reshape(_NSEG)


# speedup vs baseline: 14.2937x; 14.2937x over previous
import functools

import jax
import jax.numpy as jnp
from jax import lax
from jax.experimental import pallas as pl
from jax.experimental.pallas import tpu as pltpu
from jax.experimental.pallas import tpu_sc as plsc

_NSEG = 1024
_TGLOB = 0.5
_NC = 2
_NS = 16
_NW = _NC * _NS
_L = 16
_BLK = 8192


def _pad_to(x, n, val):
    if x.shape[0] == n:
        return x
    return jnp.concatenate([x, jnp.full((n - x.shape[0],), val, x.dtype)])


def _sc_partials(coeffs, bfi, cb, ai, ab, g_z, t_z, n_basis_p, n_atoms_p):
    b_chunk = n_basis_p // _NW
    a_chunk = n_atoms_p // _NW
    n_full, rem = divmod(b_chunk, _BLK)
    assert rem % _L == 0 and a_chunk % _L == 0 and a_chunk <= _BLK

    mesh = plsc.VectorSubcoreMesh(core_axis_name="c", subcore_axis_name="s")

    @functools.partial(
        pl.kernel,
        mesh=mesh,
        compiler_params=pltpu.CompilerParams(needs_layout_passes=False),
        out_type=jax.ShapeDtypeStruct((_NW, _NSEG), jnp.float32),
        scratch_types=[
            pltpu.VMEM((64,), jnp.float32),
            pltpu.VMEM((16,), jnp.float32),
            pltpu.VMEM((_NSEG,), jnp.float32),
            pltpu.VMEM((_BLK,), jnp.float32),
            pltpu.VMEM((_BLK,), jnp.int32),
            pltpu.VMEM((_BLK,), jnp.int32),
            pltpu.VMEM((_BLK,), jnp.int32),
            pltpu.VMEM((_BLK,), jnp.int32),
        ],
    )
    def sc_kern(coeffs_hbm, bfi_hbm, cb_hbm, ai_hbm, ab_hbm, gz_hbm, tz_hbm,
                out_hbm, g_tab, t_tab, acc, c_v, b_v, s_v, ai_v, ab_v):
        wid = lax.axis_index("s") * _NC + lax.axis_index("c")

        pltpu.sync_copy(gz_hbm, g_tab)
        pltpu.sync_copy(tz_hbm, t_tab)

        def zero_step(i, carry):
            acc[pl.ds(i * _L, _L)] = jnp.zeros((_L,), jnp.float32)
            return carry
        lax.fori_loop(0, _NSEG // _L, zero_step, 0)

        def basis_vecs(j, carry):
            c = c_v[pl.ds(j * _L, _L)]
            b = b_v[pl.ds(j * _L, _L)]
            s = s_v[pl.ds(j * _L, _L)]
            g = plsc.load_gather(g_tab, [b])
            plsc.addupdate_scatter(acc, [s], g * c)
            return carry

        base = wid * b_chunk
        for blk in range(n_full):
            off = base + blk * _BLK
            pltpu.sync_copy(coeffs_hbm.at[pl.ds(off, _BLK)], c_v)
            pltpu.sync_copy(bfi_hbm.at[pl.ds(off, _BLK)], b_v)
            pltpu.sync_copy(cb_hbm.at[pl.ds(off, _BLK)], s_v)
            lax.fori_loop(0, _BLK // _L, basis_vecs, 0)
        if rem:
            off = base + n_full * _BLK
            pltpu.sync_copy(coeffs_hbm.at[pl.ds(off, rem)], c_v.at[pl.ds(0, rem)])
            pltpu.sync_copy(bfi_hbm.at[pl.ds(off, rem)], b_v.at[pl.ds(0, rem)])
            pltpu.sync_copy(cb_hbm.at[pl.ds(off, rem)], s_v.at[pl.ds(0, rem)])
            lax.fori_loop(0, rem // _L, basis_vecs, 0)

        a_base = wid * a_chunk
        pltpu.sync_copy(ai_hbm.at[pl.ds(a_base, a_chunk)], ai_v.at[pl.ds(0, a_chunk)])
        pltpu.sync_copy(ab_hbm.at[pl.ds(a_base, a_chunk)], ab_v.at[pl.ds(0, a_chunk)])

        def atom_vecs(j, carry):
            i = ai_v[pl.ds(j * _L, _L)]
            sgm = ab_v[pl.ds(j * _L, _L)]
            t = plsc.load_gather(t_tab, [i])
            plsc.addupdate_scatter(acc, [sgm], t)
            return carry
        lax.fori_loop(0, a_chunk // _L, atom_vecs, 0)

        pltpu.sync_copy(acc, out_hbm.at[wid])

    return sc_kern(coeffs, bfi, cb, ai, ab, g_z, t_z)


def _combine_body(p_ref, o_ref):
    o_ref[...] = jnp.sum(p_ref[...], axis=0, keepdims=True) + _TGLOB


def kernel(atom_ind, coeffs, basis_function_ind, atom_batch, coeffs_batch,
           t_z, g_z):
    n_basis = coeffs.shape[0]
    n_atoms = atom_ind.shape[0]
    grain = _NW * _L
    n_basis_p = -(-n_basis // grain) * grain
    n_atoms_p = -(-n_atoms // grain) * grain

    coeffs_p = _pad_to(coeffs, n_basis_p, 0.0)
    bfi_p = _pad_to(basis_function_ind, n_basis_p, 0)
    cb_p = _pad_to(coeffs_batch, n_basis_p, 0)
    ai_p = _pad_to(atom_ind, n_atoms_p, t_z.shape[0])
    ab_p = _pad_to(atom_batch, n_atoms_p, 0)
    gz_p = _pad_to(g_z, 64, 0.0)
    tz_p = _pad_to(t_z, 16, 0.0)

    partials = _sc_partials(coeffs_p, bfi_p, cb_p, ai_p, ab_p, gz_p, tz_p,
                            n_basis_p, n_atoms_p)

    out = pl.pallas_call(
        _combine_body,
        out_shape=jax.ShapeDtypeStruct((1, _NSEG), jnp.float32),
    )(partials)
    return out.reshape(_NSEG)


# speedup vs baseline: 23.9433x; 1.6751x over previous
import functools

import jax
import jax.numpy as jnp
from jax import lax
from jax.experimental import pallas as pl
from jax.experimental.pallas import tpu as pltpu
from jax.experimental.pallas import tpu_sc as plsc

_NSEG = 1024
_TGLOB = 0.5
_NC = 2
_NS = 16
_NW = _NC * _NS
_L = 16
_BLK = 16384
_UNROLL = 8


def _pad_to(x, n, val):
    if x.shape[0] == n:
        return x
    return jnp.concatenate([x, jnp.full((n - x.shape[0],), val, x.dtype)])


def _sc_partials(coeffs, bfi, cb, ai, ab, g_z, t_z, n_basis_p, n_atoms_p):
    b_chunk = n_basis_p // _NW
    a_chunk = n_atoms_p // _NW
    n_full, rem = divmod(b_chunk, _BLK)
    assert rem % 128 == 0 and a_chunk % 128 == 0 and a_chunk <= _BLK

    mesh = plsc.VectorSubcoreMesh(core_axis_name="c", subcore_axis_name="s")

    @functools.partial(
        pl.kernel,
        mesh=mesh,
        compiler_params=pltpu.CompilerParams(needs_layout_passes=False),
        out_type=jax.ShapeDtypeStruct((_NW, _NSEG * _L), jnp.float32),
        scratch_types=[
            pltpu.VMEM((64,), jnp.float32),
            pltpu.VMEM((16,), jnp.float32),
            pltpu.VMEM((_NSEG * _L,), jnp.float32),
            pltpu.VMEM((_BLK,), jnp.float32),
            pltpu.VMEM((_BLK,), jnp.float32),
            pltpu.VMEM((_BLK,), jnp.int32),
            pltpu.VMEM((_BLK,), jnp.int32),
            pltpu.VMEM((_BLK,), jnp.int32),
            pltpu.VMEM((_BLK,), jnp.int32),
            pltpu.VMEM((a_chunk,), jnp.int32),
            pltpu.VMEM((a_chunk,), jnp.int32),
            pltpu.SemaphoreType.DMA,
            pltpu.SemaphoreType.DMA,
            pltpu.SemaphoreType.DMA,
        ],
    )
    def sc_kern(coeffs_hbm, bfi_hbm, cb_hbm, ai_hbm, ab_hbm, gz_hbm, tz_hbm,
                out_hbm, g_tab, t_tab, acc, c_0, c_1, b_0, b_1, s_0, s_1,
                ai_v, ab_v, sem0, sem1, sem_a):
        wid = lax.axis_index("s") * _NC + lax.axis_index("c")
        base = wid * b_chunk
        a_base = wid * a_chunk
        sems = (sem0, sem1)
        bufs = ((c_0, b_0, s_0), (c_1, b_1, s_1))
        lane = lax.iota(jnp.int32, _L)

        def start_block(blk, size):
            par = blk % 2
            cv, bv, sv = bufs[par]
            off = base + blk * _BLK
            return (
                pltpu.async_copy(coeffs_hbm.at[pl.ds(off, size)],
                                 cv.at[pl.ds(0, size)], sems[par]),
                pltpu.async_copy(bfi_hbm.at[pl.ds(off, size)],
                                 bv.at[pl.ds(0, size)], sems[par]),
                pltpu.async_copy(cb_hbm.at[pl.ds(off, size)],
                                 sv.at[pl.ds(0, size)], sems[par]),
            )

        n_blocks = n_full + (1 if rem else 0)
        sizes = [_BLK] * n_full + ([rem] if rem else [])

        d_ai = pltpu.async_copy(ai_hbm.at[pl.ds(a_base, a_chunk)], ai_v, sem_a)
        d_ab = pltpu.async_copy(ab_hbm.at[pl.ds(a_base, a_chunk)], ab_v, sem_a)
        d_gz = pltpu.async_copy(gz_hbm, g_tab, sem_a)
        d_tz = pltpu.async_copy(tz_hbm, t_tab, sem_a)
        pending = start_block(0, sizes[0]) if n_blocks else ()

        @plsc.parallel_loop(0, _NSEG * _L, _L, unroll=_UNROLL)
        def _zero(i):
            acc[pl.ds(i, _L)] = jnp.zeros((_L,), jnp.float32)

        def compute_block(par, nvec, unroll):
            cv, bv, sv = bufs[par]

            @plsc.parallel_loop(0, nvec * _L, _L, unroll=unroll)
            def _body(i):
                c = cv[pl.ds(i, _L)]
                b = bv[pl.ds(i, _L)]
                s = sv[pl.ds(i, _L)]
                g = plsc.load_gather(g_tab, [b])
                plsc.addupdate_scatter(acc, [s * _L + lane], g * c)

        for blk in range(n_blocks):
            for d in pending:
                d.wait()
            if blk + 1 < n_blocks:
                nxt = start_block(blk + 1, sizes[blk + 1])
            nvec = sizes[blk] // _L
            compute_block(blk % 2, nvec, _UNROLL if nvec % _UNROLL == 0 else 1)
            if blk + 1 < n_blocks:
                pending = nxt

        d_ai.wait(); d_ab.wait(); d_gz.wait(); d_tz.wait()

        @plsc.parallel_loop(0, a_chunk, _L,
                            unroll=_UNROLL if (a_chunk // _L) % _UNROLL == 0 else 1)
        def _atom(i):
            idx = ai_v[pl.ds(i, _L)]
            sgm = ab_v[pl.ds(i, _L)]
            t = plsc.load_gather(t_tab, [idx])
            plsc.addupdate_scatter(acc, [sgm * _L + lane], t)

        pltpu.sync_copy(acc, out_hbm.at[wid])

    return sc_kern(coeffs, bfi, cb, ai, ab, g_z, t_z)


def _combine_body(p_ref, o_ref):
    p = p_ref[...]
    o_ref[...] = jnp.sum(p, axis=(0, 2)).reshape(1, _NSEG) + _TGLOB


def kernel(atom_ind, coeffs, basis_function_ind, atom_batch, coeffs_batch,
           t_z, g_z):
    n_basis = coeffs.shape[0]
    n_atoms = atom_ind.shape[0]
    grain = _NW * 128
    n_basis_p = -(-n_basis // grain) * grain
    n_atoms_p = -(-n_atoms // grain) * grain

    coeffs_p = _pad_to(coeffs, n_basis_p, 0.0)
    bfi_p = _pad_to(basis_function_ind, n_basis_p, 0)
    cb_p = _pad_to(coeffs_batch, n_basis_p, 0)
    ai_p = _pad_to(atom_ind, n_atoms_p, t_z.shape[0])
    ab_p = _pad_to(atom_batch, n_atoms_p, 0)
    gz_p = _pad_to(g_z, 64, 0.0)
    tz_p = _pad_to(t_z, 16, 0.0)

    partials = _sc_partials(coeffs_p, bfi_p, cb_p, ai_p, ab_p, gz_p, tz_p,
                            n_basis_p, n_atoms_p)
    partials = partials.reshape(_NW, _NSEG, _L)

    out = pl.pallas_call(
        _combine_body,
        out_shape=jax.ShapeDtypeStruct((1, _NSEG), jnp.float32),
    )(partials)
    return out.reshape(_NSEG)
